# baseline (speedup 1.0000x reference)
import jax
import jax.numpy as jnp
from jax import lax
from jax.experimental import pallas as pl
from jax.experimental.pallas import tpu as pltpu
from jax.experimental.pallas import tpu_sc as plsc

MAX_LEN = 2048
D_K = 32
NSHIFT = 128
TBW = 2 * MAX_LEN
PADW = TBW + NSHIFT

_info = plsc.get_sparse_core_info()
_NC, _NS = _info.num_cores, _info.num_subcores
_NW = _NC * _NS
RES_PER_W = NSHIFT // _NW
ROWS_PER_RES = MAX_LEN // NSHIFT
WIN_W = (ROWS_PER_RES - 1) * NSHIFT + MAX_LEN
HALF_K = D_K // 2


def _tc_build_body(big_ref, out_ref):
    d = pl.program_id(0)
    rolled = pltpu.roll(big_ref[:, :], lax.rem(PADW - d, PADW), 1)
    out_ref[0] = rolled[:, :TBW]


def _sc_body(tablesT_hbm, out_hbm, win_v, isem, osem):
    wid = lax.axis_index("s") * _NC + lax.axis_index("c")
    nhalf = RES_PER_W * 2

    def _stage(h, slot):
        delta = wid * RES_PER_W + h // 2
        kk = (h % 2) * HALF_K
        return pltpu.async_copy(
            tablesT_hbm.at[delta, pl.ds(kk, HALF_K), pl.ds(0, WIN_W)],
            win_v.at[slot],
            isem,
        )

    _stage(0, 0).wait()
    for h in range(nhalf):
        slot = h % 2
        if h + 1 < nhalf:
            nxt = _stage(h + 1, 1 - slot)
        delta = wid * RES_PER_W + h // 2
        kk = (h % 2) * HALF_K
        copies = []
        for m in range(ROWS_PER_RES):
            copies.append(
                pltpu.async_copy(
                    win_v.at[slot, :, pl.ds((ROWS_PER_RES - 1 - m) * NSHIFT, MAX_LEN)],
                    out_hbm.at[(NSHIFT - 1) - delta + NSHIFT * m, pl.ds(kk, HALF_K)],
                    osem,
                )
            )
        for c in copies:
            c.wait()
        if h + 1 < nhalf:
            nxt.wait()


@jax.jit
def _run(rel_embeddings):
    tableT = rel_embeddings.T
    bigpad = jnp.pad(tableT, ((0, 0), (0, PADW - (2 * MAX_LEN - 1))))
    tablesT = pl.pallas_call(
        _tc_build_body,
        grid=(NSHIFT,),
        in_specs=[pl.BlockSpec((D_K, PADW), lambda d: (0, 0))],
        out_specs=pl.BlockSpec((1, D_K, TBW), lambda d: (d, 0, 0)),
        out_shape=jax.ShapeDtypeStruct((NSHIFT, D_K, TBW), jnp.float32),
    )(bigpad)
    k = pl.kernel(
        _sc_body,
        out_type=jax.ShapeDtypeStruct((MAX_LEN, D_K, MAX_LEN), jnp.float32),
        mesh=plsc.VectorSubcoreMesh(core_axis_name="c", subcore_axis_name="s"),
        scratch_types=[
            pltpu.VMEM((2, HALF_K, WIN_W), jnp.float32),
            pltpu.SemaphoreType.DMA,
            pltpu.SemaphoreType.DMA,
        ],
    )
    out_phys = k(tablesT)
    return jnp.transpose(out_phys, (0, 2, 1))


def kernel(seq_len, rel_embeddings):
    del seq_len
    return _run(rel_embeddings)


# speedup vs baseline: 1.0213x; 1.0213x over previous
import jax
import jax.numpy as jnp
from jax import lax
from jax.experimental import pallas as pl
from jax.experimental.pallas import tpu as pltpu
from jax.experimental.pallas import tpu_sc as plsc

MAX_LEN = 2048
D_K = 32
NSHIFT = 128
TBW = 2 * MAX_LEN
PADW = TBW + NSHIFT

_info = plsc.get_sparse_core_info()
_NC, _NS = _info.num_cores, _info.num_subcores
_NW = _NC * _NS
RES_PER_W = NSHIFT // _NW
ROWS_PER_RES = MAX_LEN // NSHIFT
WIN_W = (ROWS_PER_RES - 1) * NSHIFT + MAX_LEN


def _tc_build_body(big_ref, out_ref):
    d = pl.program_id(0)
    rolled = pltpu.roll(big_ref[:, :], lax.rem(PADW - d, PADW), 1)
    out_ref[0] = rolled[:, :TBW]


def _sc_body(tablesT_hbm, out_hbm, win_v, isem, osem):
    wid = lax.axis_index("s") * _NC + lax.axis_index("c")
    for dd in range(RES_PER_W):
        delta = wid * RES_PER_W + dd
        pltpu.async_copy(
            tablesT_hbm.at[delta, :, pl.ds(0, WIN_W)], win_v, isem
        ).wait()
        copies = []
        for m in range(ROWS_PER_RES):
            copies.append(
                pltpu.async_copy(
                    win_v.at[:, pl.ds((ROWS_PER_RES - 1 - m) * NSHIFT, MAX_LEN)],
                    out_hbm.at[(NSHIFT - 1) - delta + NSHIFT * m],
                    osem,
                )
            )
        for c in copies:
            c.wait()


@jax.jit
def _run(rel_embeddings):
    tableT = rel_embeddings.T
    bigpad = jnp.pad(tableT, ((0, 0), (0, PADW - (2 * MAX_LEN - 1))))
    tablesT = pl.pallas_call(
        _tc_build_body,
        grid=(NSHIFT,),
        in_specs=[pl.BlockSpec((D_K, PADW), lambda d: (0, 0))],
        out_specs=pl.BlockSpec((1, D_K, TBW), lambda d: (d, 0, 0)),
        out_shape=jax.ShapeDtypeStruct((NSHIFT, D_K, TBW), jnp.float32),
    )(bigpad)
    k = pl.kernel(
        _sc_body,
        out_type=jax.ShapeDtypeStruct((MAX_LEN, D_K, MAX_LEN), jnp.float32),
        mesh=plsc.VectorSubcoreMesh(core_axis_name="c", subcore_axis_name="s"),
        scratch_types=[
            pltpu.VMEM((D_K, WIN_W), jnp.float32),
            pltpu.SemaphoreType.DMA,
            pltpu.SemaphoreType.DMA,
        ],
    )
    out_phys = k(tablesT)
    return jnp.transpose(out_phys, (0, 2, 1))


def kernel(seq_len, rel_embeddings):
    del seq_len
    return _run(rel_embeddings)
